# baseline (speedup 1.0000x reference)
import functools

import jax
import jax.numpy as jnp
from jax import lax
from jax.experimental import pallas as pl
from jax.experimental.pallas import tpu as pltpu
from jax.experimental.pallas import tpu_sc as plsc

IN_DIM = 768
HASHES = 2
BITS = 6
NGROUPS = 2 * HASHES
NPROJ = NGROUPS * BITS
OUT_LANES = 8
ROWS = 4096

NC, NS, L = 2, 16, 16
NW = NC * NS
KCH = IN_DIM // L
CH = 16
N_SC = 512


def _router_body(x_ref, wt_ref, aux_ref, out_ref):
    z = jnp.dot(x_ref[...], wt_ref[...], preferred_element_type=jnp.float32)
    bits = (z > 0).astype(jnp.float32)
    j = lax.broadcasted_iota(jnp.int32, (128, OUT_LANES), 0)
    g = lax.broadcasted_iota(jnp.int32, (128, OUT_LANES), 1)
    mask = (j < BITS * NGROUPS) & ((j // BITS) == g)
    s = jnp.where(mask, (1 << (j % BITS)).astype(jnp.float32), 0.0)
    packed = jnp.dot(bits, s, preferred_element_type=jnp.float32)
    idx = packed.astype(jnp.int32) + aux_ref[0, 0]
    out_ref[...] = idx.T


def _router(x, wt, aux, n_tc):
    grid = (pl.cdiv(n_tc, ROWS),)
    return pl.pallas_call(
        _router_body,
        grid=grid,
        in_specs=[
            pl.BlockSpec((ROWS, IN_DIM), lambda i: (i, 0)),
            pl.BlockSpec((IN_DIM, 128), lambda i: (0, 0)),
            pl.BlockSpec((1, 1), lambda i: (0, 0), memory_space=pltpu.MemorySpace.SMEM),
        ],
        out_specs=pl.BlockSpec((OUT_LANES, ROWS), lambda i: (0, i)),
        out_shape=jax.ShapeDtypeStruct((OUT_LANES, n_tc), jnp.int32),
        compiler_params=pltpu.CompilerParams(
            dimension_semantics=("arbitrary",),
        ),
    )(x, wt, aux)


def _rne_bf16(v):
    r = lax.bitcast_convert_type(v, jnp.int32)
    r = r + jnp.int32(0x7FFF) + ((r >> 16) & jnp.int32(1))
    r = r & jnp.int32(-65536)
    return lax.bitcast_convert_type(r, jnp.float32)


def _sc_body(start_row, n_rows, x_hbm, w_hbm, aux_hbm, out_hbm, w_v, aux_v, xb, ob):
    cid = lax.axis_index("c")
    sid = lax.axis_index("s")
    wid = sid * NC + cid
    rows_w = n_rows // NW
    base = start_row + wid * rows_w
    out_base = wid * rows_w
    pltpu.sync_copy(w_hbm, w_v)
    pltpu.sync_copy(aux_hbm, aux_v)
    lane = lax.broadcasted_iota(jnp.int32, (L,), 0)

    def chunk_body(c, carry):
        row0 = base + c * CH
        pltpu.sync_copy(x_hbm.at[pl.ds(row0 * IN_DIM, CH * IN_DIM)], xb)

        RU = 8

        def round_body(i, rcarry):
            for u in range(RU):
                off = (i * RU + u) * L
                xb[pl.ds(off, L)] = _rne_bf16(xb[pl.ds(off, L)])
            return rcarry

        lax.fori_loop(0, CH * KCH // RU, round_body, 0)

        def pair_body(p, tcarry):
            x0off = (2 * p) * IN_DIM
            x1off = x0off + IN_DIM
            out0 = aux_v[...]
            out1 = aux_v[...]
            for g in range(NGROUPS):
                accs = [jnp.zeros((L,), jnp.float32) for _ in range(2 * BITS)]
                for k in range(KCH):
                    xv0 = xb[pl.ds(x0off + k * L, L)]
                    xv1 = xb[pl.ds(x1off + k * L, L)]
                    for bb in range(BITS):
                        wv = w_v[pl.ds(((g * BITS + bb) * KCH + k) * L, L)]
                        accs[2 * bb] = accs[2 * bb] + wv * xv0
                        accs[2 * bb + 1] = accs[2 * bb + 1] + wv * xv1
                s0 = jnp.zeros((L,), jnp.int32)
                s1 = jnp.zeros((L,), jnp.int32)
                for bb in range(BITS):
                    v0 = accs[2 * bb]
                    v1 = accs[2 * bb + 1]
                    for st in (8, 4, 2, 1):
                        v0 = v0 + jnp.take_along_axis(v0, lane ^ st, axis=0)
                        v1 = v1 + jnp.take_along_axis(v1, lane ^ st, axis=0)
                    s0 = s0 + jnp.where(v0 > 0, jnp.int32(1 << bb), jnp.int32(0))
                    s1 = s1 + jnp.where(v1 > 0, jnp.int32(1 << bb), jnp.int32(0))
                gmask = lane == g
                out0 = out0 + jnp.where(gmask, s0, jnp.int32(0))
                out1 = out1 + jnp.where(gmask, s1, jnp.int32(0))
            ob[pl.ds((2 * p) * L, L)] = out0
            ob[pl.ds((2 * p + 1) * L, L)] = out1
            return tcarry

        lax.fori_loop(0, CH // 2, pair_body, 0)
        pltpu.sync_copy(ob, out_hbm.at[pl.ds((out_base + c * CH) * L, CH * L)])
        return carry

    lax.fori_loop(0, rows_w // CH, chunk_body, 0)


def _sc_router(x_flat, w_flat, aux_arr, start_row, n_rows):
    mesh = plsc.VectorSubcoreMesh(
        core_axis_name="c", subcore_axis_name="s", num_cores=NC, num_subcores=NS)
    return pl.kernel(
        functools.partial(_sc_body, start_row, n_rows),
        out_type=jax.ShapeDtypeStruct((n_rows * L,), jnp.int32),
        mesh=mesh,
        scratch_types=[
            pltpu.VMEM((NPROJ * IN_DIM,), jnp.float32),
            pltpu.VMEM((L,), jnp.int32),
            pltpu.VMEM((CH * IN_DIM,), jnp.float32),
            pltpu.VMEM((CH * L,), jnp.int32),
        ],
    )(x_flat, w_flat, aux_arr)


def kernel(tag, W_read, W_write, collect_aux=0):
    B, T, D = tag.shape
    n = B * T
    n_tc = n - N_SC
    x = tag.reshape(n, D)
    w = jnp.concatenate([W_read, W_write], axis=0)
    wt = jnp.pad(w.T, ((0, 0), (0, 128 - NPROJ)))
    aux = jnp.asarray(collect_aux, dtype=jnp.int32).reshape(1, 1)
    w_flat = w.astype(jnp.bfloat16).astype(jnp.float32).reshape(-1)
    aux_arr = jnp.full((L,), jnp.asarray(collect_aux, jnp.int32))

    out_sc = _sc_router(x.reshape(-1), w_flat, aux_arr, n_tc, N_SC)
    out_tc = _router(x, wt, aux, n_tc)

    sc = out_sc.reshape(N_SC, L)
    idx_r = jnp.concatenate(
        [out_tc[0:HASHES].T, sc[:, 0:HASHES]], axis=0).reshape(B, T, HASHES)
    idx_w = jnp.concatenate(
        [out_tc[HASHES:2 * HASHES].T, sc[:, HASHES:2 * HASHES]], axis=0).reshape(B, T, HASHES)
    return idx_r, idx_w


# speedup vs baseline: 3.6958x; 3.6958x over previous
import jax
import jax.numpy as jnp
from jax import lax
from jax.experimental import pallas as pl
from jax.experimental.pallas import tpu as pltpu

IN_DIM = 768
HASHES = 2
BITS = 6
NGROUPS = 2 * HASHES
OUT_LANES = 8
ROWS = 8192


def _router_body(x_ref, wt_ref, aux_ref, out_ref):
    z = jnp.dot(x_ref[...], wt_ref[...], preferred_element_type=jnp.float32)
    bits = (z > 0).astype(jnp.float32)
    j = lax.broadcasted_iota(jnp.int32, (128, OUT_LANES), 0)
    g = lax.broadcasted_iota(jnp.int32, (128, OUT_LANES), 1)
    mask = (j < BITS * NGROUPS) & ((j // BITS) == g)
    s = jnp.where(mask, (1 << (j % BITS)).astype(jnp.float32), 0.0)
    packed = jnp.dot(bits, s, preferred_element_type=jnp.float32)
    idx = packed.astype(jnp.int32) + aux_ref[0, 0]
    out_ref[...] = idx.T


def _router(x, wt, aux):
    n = x.shape[0]
    grid = (n // ROWS,)
    return pl.pallas_call(
        _router_body,
        grid=grid,
        in_specs=[
            pl.BlockSpec((ROWS, IN_DIM), lambda i: (i, 0)),
            pl.BlockSpec((IN_DIM, 128), lambda i: (0, 0)),
            pl.BlockSpec((1, 1), lambda i: (0, 0), memory_space=pltpu.MemorySpace.SMEM),
        ],
        out_specs=pl.BlockSpec((OUT_LANES, ROWS), lambda i: (0, i)),
        out_shape=jax.ShapeDtypeStruct((OUT_LANES, n), jnp.int32),
        compiler_params=pltpu.CompilerParams(
            dimension_semantics=("arbitrary",),
        ),
    )(x, wt, aux)


def kernel(tag, W_read, W_write, collect_aux=0):
    B, T, D = tag.shape
    x = tag.reshape(B * T, D)
    w = jnp.concatenate([W_read, W_write], axis=0)
    wt = jnp.pad(w.T, ((0, 0), (0, 128 - NGROUPS * BITS)))
    aux = jnp.asarray(collect_aux, dtype=jnp.int32).reshape(1, 1)
    out = _router(x, wt, aux)
    idx_r = out[0:HASHES].T.reshape(B, T, HASHES)
    idx_w = out[HASHES:2 * HASHES].T.reshape(B, T, HASHES)
    return idx_r, idx_w


# speedup vs baseline: 3.9846x; 1.0781x over previous
import jax
import jax.numpy as jnp
from jax import lax
from jax.experimental import pallas as pl
from jax.experimental.pallas import tpu as pltpu

IN_DIM = 768
HASHES = 2
BITS = 6
NGROUPS = 2 * HASHES
OUT_LANES = 8
ROWS = 4096


def _router_body(x_ref, wt_ref, aux_ref, out_ref):
    z = jnp.dot(x_ref[...], wt_ref[...], preferred_element_type=jnp.float32)
    bits = (z > 0).astype(jnp.float32)
    j = lax.broadcasted_iota(jnp.int32, (128, OUT_LANES), 0)
    g = lax.broadcasted_iota(jnp.int32, (128, OUT_LANES), 1)
    mask = (j < BITS * NGROUPS) & ((j // BITS) == g)
    s = jnp.where(mask, (1 << (j % BITS)).astype(jnp.float32), 0.0)
    packed = jnp.dot(bits, s, preferred_element_type=jnp.float32)
    idx = packed.astype(jnp.int32) + aux_ref[0, 0]
    out_ref[...] = idx.T


def _router(x, wt, aux):
    n = x.shape[0]
    grid = (n // ROWS,)
    return pl.pallas_call(
        _router_body,
        grid=grid,
        in_specs=[
            pl.BlockSpec((ROWS, IN_DIM), lambda i: (i, 0)),
            pl.BlockSpec((IN_DIM, 128), lambda i: (0, 0)),
            pl.BlockSpec((1, 1), lambda i: (0, 0), memory_space=pltpu.MemorySpace.SMEM),
        ],
        out_specs=pl.BlockSpec((OUT_LANES, ROWS), lambda i: (0, i)),
        out_shape=jax.ShapeDtypeStruct((OUT_LANES, n), jnp.int32),
        compiler_params=pltpu.CompilerParams(
            dimension_semantics=("arbitrary",),
        ),
    )(x, wt, aux)


def kernel(tag, W_read, W_write, collect_aux=0):
    B, T, D = tag.shape
    x = tag.reshape(B * T, D)
    w = jnp.concatenate([W_read, W_write], axis=0)
    wt = jnp.pad(w.T, ((0, 0), (0, 128 - NGROUPS * BITS)))
    aux = jnp.asarray(collect_aux, dtype=jnp.int32).reshape(1, 1)
    out = _router(x, wt, aux)
    idx_r = out[0:HASHES].T.reshape(B, T, HASHES)
    idx_w = out[HASHES:2 * HASHES].T.reshape(B, T, HASHES)
    return idx_r, idx_w
